# baseline (speedup 1.0000x reference)
import functools

import jax
import jax.numpy as jnp
from jax import lax
from jax.experimental import pallas as pl
from jax.experimental.pallas import tpu as pltpu
from jax.experimental.pallas import tpu_sc as plsc

_B = 8
_KV_LEN = 8192
_D = 1024
_NC = 2
_NS = 16
_NW = _NC * _NS
_CHUNK = (_B * _D) // _NW
_CPB = _D // _CHUNK

_mesh = plsc.VectorSubcoreMesh(
    core_axis_name="c", subcore_axis_name="s", num_cores=1
)


@functools.partial(
    pl.kernel,
    mesh=_mesh,
    out_type=jax.ShapeDtypeStruct((_B, 1, _D), jnp.float32),
)
def _write_narrowed(token_hbm, out_hbm):
    sid = lax.axis_index("s")
    half = _B // _NS if _B >= _NS else 1

    @pl.when(sid < _B)
    def _():
        pltpu.sync_copy(token_hbm.at[pl.ds(sid * half, half)],
                        out_hbm.at[pl.ds(sid * half, half)])


def kernel(input_token, input_pos, kv_cache):
    del input_pos, kv_cache
    return _write_narrowed(input_token)


# speedup vs baseline: 1.0939x; 1.0939x over previous
import functools

import jax
import jax.numpy as jnp
from jax.experimental import pallas as pl
from jax.experimental.pallas import tpu as pltpu
from jax.experimental.pallas import tpu_sc as plsc

_B = 8
_D = 1024

_mesh = plsc.ScalarSubcoreMesh(axis_name="c", num_cores=1)


@functools.partial(
    pl.kernel,
    mesh=_mesh,
    out_type=jax.ShapeDtypeStruct((_B, 1, _D), jnp.float32),
)
def _write_narrowed(token_hbm, out_hbm):
    pltpu.sync_copy(token_hbm, out_hbm)


def kernel(input_token, input_pos, kv_cache):
    del input_pos, kv_cache
    return _write_narrowed(input_token)


# speedup vs baseline: 1.0947x; 1.0008x over previous
import functools

import jax
import jax.numpy as jnp
from jax.experimental import pallas as pl
from jax.experimental.pallas import tpu as pltpu
from jax.experimental.pallas import tpu_sc as plsc

_B = 8
_D = 1024

_mesh = plsc.ScalarSubcoreMesh(axis_name="c", num_cores=1)


@functools.partial(
    pl.kernel,
    mesh=_mesh,
    out_type=jax.ShapeDtypeStruct((_B, 1, _D), jnp.float32),
    scratch_types=[pltpu.SemaphoreType.DMA, pltpu.SemaphoreType.DMA],
)
def _write_narrowed(token_hbm, out_hbm, sem0, sem1):
    half = _B // 2
    c0 = pltpu.async_copy(token_hbm.at[pl.ds(0, half)],
                          out_hbm.at[pl.ds(0, half)], sem0)
    c1 = pltpu.async_copy(token_hbm.at[pl.ds(half, half)],
                          out_hbm.at[pl.ds(half, half)], sem1)
    c0.wait()
    c1.wait()


def kernel(input_token, input_pos, kv_cache):
    del input_pos, kv_cache
    return _write_narrowed(input_token)
